# baseline (speedup 1.0000x reference)
import math
import functools

import jax
import jax.numpy as jnp
from jax.experimental import pallas as pl
from jax.experimental.pallas import tpu as pltpu

_DEPTH = 4


def _rup(n, m):
    return ((n + m - 1) // m) * m


def _mm_body(a_ref, b_ref, o_ref, acc_ref, *, nsteps):
    k = pl.program_id(2)

    @pl.when(k == 0)
    def _():
        acc_ref[...] = jnp.zeros_like(acc_ref)

    acc_ref[...] += jnp.dot(a_ref[...], b_ref[...],
                            preferred_element_type=jnp.float32)

    @pl.when(k == nsteps - 1)
    def _():
        o_ref[...] = acc_ref[...]


def _mm(a, b, bm=256, bn=128, bk=256):
    m, ka = a.shape
    kb, n = b.shape
    assert ka == kb and m % bm == 0 and n % bn == 0 and ka % bk == 0
    grid = (m // bm, n // bn, ka // bk)
    return pl.pallas_call(
        functools.partial(_mm_body, nsteps=grid[2]),
        grid=grid,
        in_specs=[
            pl.BlockSpec((bm, bk), lambda i, j, k: (i, k)),
            pl.BlockSpec((bk, bn), lambda i, j, k: (k, j)),
        ],
        out_specs=pl.BlockSpec((bm, bn), lambda i, j, k: (i, j)),
        out_shape=jax.ShapeDtypeStruct((m, n), jnp.float32),
        scratch_shapes=[pltpu.VMEM((bm, bn), jnp.float32)],
        compiler_params=pltpu.CompilerParams(
            dimension_semantics=("parallel", "parallel", "arbitrary")),
    )(a, b)


def _mm_diag0_body(a_ref, b_ref, o_ref, acc_ref, *, nsteps, bm, bn):
    k = pl.program_id(2)
    i = pl.program_id(0)
    j = pl.program_id(1)

    @pl.when(k == 0)
    def _():
        acc_ref[...] = jnp.zeros_like(acc_ref)

    acc_ref[...] += jnp.dot(a_ref[...], b_ref[...],
                            preferred_element_type=jnp.float32)

    @pl.when(k == nsteps - 1)
    def _():
        ri = jax.lax.broadcasted_iota(jnp.int32, (bm, bn), 0) + i * bm
        ci = jax.lax.broadcasted_iota(jnp.int32, (bm, bn), 1) + j * bn
        o_ref[...] = jnp.where(ri == ci, 0.0, acc_ref[...])


def _mm_diag0(a, b, bm=256, bn=256, bk=256):
    m, ka = a.shape
    kb, n = b.shape
    assert ka == kb and m % bm == 0 and n % bn == 0 and ka % bk == 0
    grid = (m // bm, n // bn, ka // bk)
    return pl.pallas_call(
        functools.partial(_mm_diag0_body, nsteps=grid[2], bm=bm, bn=bn),
        grid=grid,
        in_specs=[
            pl.BlockSpec((bm, bk), lambda i, j, k: (i, k)),
            pl.BlockSpec((bk, bn), lambda i, j, k: (k, j)),
        ],
        out_specs=pl.BlockSpec((bm, bn), lambda i, j, k: (i, j)),
        out_shape=jax.ShapeDtypeStruct((m, n), jnp.float32),
        scratch_shapes=[pltpu.VMEM((bm, bn), jnp.float32)],
        compiler_params=pltpu.CompilerParams(
            dimension_semantics=("parallel", "parallel", "arbitrary")),
    )(a, b)


def _gcn_body(a_ref, z_ref, zi_ref, dinv_ref, bias_ref, o_ref, acc_ref, *,
              nsteps, relu):
    k = pl.program_id(2)

    @pl.when(k == 0)
    def _():
        acc_ref[...] = jnp.zeros_like(acc_ref)

    acc_ref[...] += jnp.dot(a_ref[...], z_ref[...],
                            preferred_element_type=jnp.float32)

    @pl.when(k == nsteps - 1)
    def _():
        out = (acc_ref[...] + 2.0 * zi_ref[...]) * dinv_ref[:, :1] \
            + bias_ref[0:1, :]
        if relu:
            out = jnp.maximum(out, 0.0)
        o_ref[...] = out


def _gcn_dense(a, z, dinv, bias, relu, bm=256, bk=256):
    m, ka = a.shape
    kb, f = z.shape
    assert ka == m and kb == m and m % bm == 0 and m % bk == 0 and f % 128 == 0
    bn = min(f, 256)
    assert f % bn == 0
    grid = (m // bm, f // bn, m // bk)
    dinv2d = jnp.broadcast_to(dinv[:, None], (m, 128))
    bias2d = jnp.broadcast_to(bias[None, :], (8, f))
    return pl.pallas_call(
        functools.partial(_gcn_body, nsteps=grid[2], relu=relu),
        grid=grid,
        in_specs=[
            pl.BlockSpec((bm, bk), lambda i, j, k: (i, k)),
            pl.BlockSpec((bk, bn), lambda i, j, k: (k, j)),
            pl.BlockSpec((bm, bn), lambda i, j, k: (i, j)),
            pl.BlockSpec((bm, 128), lambda i, j, k: (i, 0)),
            pl.BlockSpec((8, bn), lambda i, j, k: (0, j)),
        ],
        out_specs=pl.BlockSpec((bm, bn), lambda i, j, k: (i, j)),
        out_shape=jax.ShapeDtypeStruct((m, f), jnp.float32),
        scratch_shapes=[pltpu.VMEM((bm, bn), jnp.float32)],
        compiler_params=pltpu.CompilerParams(
            dimension_semantics=("parallel", "parallel", "arbitrary")),
    )(a, z, z, dinv2d, bias2d)


def _pad2(x, mp, np_):
    m, n = x.shape
    if m == mp and n == np_:
        return x
    return jnp.pad(x, ((0, mp - m), (0, np_ - n)))


def _rowsum(a_pad):
    m = a_pad.shape[1]
    ones = jnp.ones((m, 128), jnp.float32)
    return _mm(a_pad, ones, bm=256, bn=128, bk=256)[:, 0]


def _xw(x, w):
    m, ci = x.shape
    ci2, co = w.shape
    mp, cip, cop = _rup(m, 256), _rup(ci, 128), _rup(co, 128)
    out = _mm(_pad2(x, mp, cip), _pad2(w, cip, cop), bm=256, bn=128, bk=128)
    return out[:m, :co]


def _gcn_dense_level(a_pad, n, x, w, b, relu):
    np_ = a_pad.shape[0]
    deg = _rowsum(a_pad)[:n] + 2.0
    dinv = jnp.where(deg > 0, jax.lax.rsqrt(deg), 0.0)
    xw = _xw(x, w)
    co = w.shape[1]
    cop = _rup(co, 128)
    z = xw * dinv[:, None]
    z_pad = _pad2(z, np_, cop)
    dinv_pad = jnp.pad(dinv, (0, np_ - n))
    b_pad = jnp.pad(b, (0, cop - co))
    out = _gcn_dense(a_pad, z_pad, dinv_pad, b_pad, relu)
    return out[:n, :co]


def _topk_pool_x(x, p):
    score = jnp.tanh((x @ p) / jnp.linalg.norm(p))
    k = int(math.ceil(0.5 * x.shape[0]))
    vals, perm = jax.lax.top_k(score, k)
    return x[perm] * vals[:, None], vals, perm


def _pooled_square(a_pad, n, k, perm):
    np_ = a_pad.shape[0]
    idx = jnp.arange(n)
    b = a_pad.at[idx, idx].set(1.0)
    kp = _rup(k, 256)
    r = jnp.pad(b[perm, :], ((0, kp - k), (0, 0)))
    c = jnp.pad(b[:, perm], ((0, 0), (0, kp - k)))
    return _mm_diag0(r, c, bm=256, bn=256, bk=256)


def _gcn_sparse0(dst, src, n, x, w, b, relu, deg0):
    dinv = jax.lax.rsqrt(deg0)
    xw = _xw(x, w)
    z = xw * dinv[:, None]
    agg = jnp.zeros_like(z).at[dst].add(z[src])
    out = (agg + 2.0 * z) * dinv[:, None] + b[None, :]
    if relu:
        out = jnp.maximum(out, 0.0)
    return out


def kernel(x, params, edge_index):
    n, f = x.shape
    dst = edge_index[1]
    src = edge_index[0]

    deg0 = jnp.zeros((n,), jnp.float32).at[dst].add(1.0) + 2.0

    wd, bd = params['Wd'], params['bd']
    wu, bu = params['Wu'], params['bu']
    ps = params['p']

    xd0 = _gcn_sparse0(dst, src, n, x, wd[0], bd[0], True, deg0)

    x1p, vals1, perm1 = _topk_pool_x(xd0, ps[0])
    k1 = x1p.shape[0]
    k1p, np0 = _rup(k1, 256), _rup(n, 256)
    rowpos = jnp.full((n,), -1, jnp.int32).at[perm1].set(
        jnp.arange(k1, dtype=jnp.int32))
    colpos = rowpos
    notself = dst != src
    rr = rowpos[dst]
    r_rows = jnp.where(notself & (rr >= 0), rr, k1p)
    r0 = jnp.zeros((k1p, np0), jnp.float32).at[r_rows, src].add(
        1.0, mode='drop')
    r0 = r0.at[jnp.arange(k1), perm1].set(1.0)
    cc = colpos[src]
    c_cols = jnp.where(notself & (cc >= 0), cc, k1p)
    c0 = jnp.zeros((np0, k1p), jnp.float32).at[dst, c_cols].add(
        1.0, mode='drop')
    c0 = c0.at[perm1, jnp.arange(k1)].set(1.0)
    a1 = _mm_diag0(r0, c0)

    xs = [xd0]
    a_pads = [None, a1]
    ks = [n, k1]
    xp = x1p
    for i in range(1, _DEPTH + 1):
        a_pad = a_pads[i]
        ki = ks[i]
        xd = _gcn_dense_level(a_pad, ki, xp, wd[i], bd[i], True)
        if i < _DEPTH:
            xs.append(xd)
            xp, vals, perm = _topk_pool_x(xd, ps[i])
            kn = xp.shape[0]
            ks.append(kn)
            a_pads.append(_pooled_square(a_pad, ki, kn, perm))
            if i == 1:
                perm2 = perm
            elif i == 2:
                perm3 = perm
            else:
                perm4 = perm
        else:
            xcur = xd

    perms = [perm1, perm2, perm3, perm4]

    for i in range(_DEPTH):
        j = _DEPTH - 1 - i
        res = xs[j]
        perm = perms[j]
        fj = res.shape[1]
        up = jnp.zeros_like(res).at[perm].set(xcur[:, :fj])
        cat = jnp.concatenate([res, up], axis=-1)
        if j > 0:
            xcur = _gcn_dense_level(a_pads[j], ks[j], cat, wu[i], bu[i],
                                    i < _DEPTH - 1)
        else:
            xcur = _gcn_sparse0(dst, src, n, cat, wu[i], bu[i], False, deg0)

    out = _gcn_sparse0(dst, src, n, xcur, params['Wo'], params['bo'],
                       False, deg0)
    return jax.nn.softmax(out, axis=1)
